# baseline (speedup 1.0000x reference)
import functools

import jax
import jax.numpy as jnp
from jax import lax
from jax.experimental import pallas as pl
from jax.experimental.pallas import tpu as pltpu
from jax.experimental.pallas import tpu_sc as plsc

_RATIO = 0.2


def _tf2x32(k0, k1, c0, c1):
    u = jnp.uint32
    ks2 = k0 ^ k1 ^ u(0x1BD11BDA)
    ks = (k0, k1, ks2)
    x0 = c0 + k0
    x1 = c1 + k1
    rots = ((13, 15, 26, 6), (17, 29, 16, 24))
    for i in range(5):
        for r in rots[i % 2]:
            x0 = x0 + x1
            x1 = ((x1 << u(r)) | (x1 >> u(32 - r))) ^ x0
        x0 = x0 + ks[(i + 1) % 3]
        x1 = x1 + ks[(i + 2) % 3] + u(i + 1)
    return x0, x1


def _randint_mod(k0, k1, span):
    u = jnp.uint32
    a0, a1 = _tf2x32(k0, k1, u(0), u(0))
    b0, b1 = _tf2x32(k0, k1, u(0), u(1))
    h0, h1 = _tf2x32(a0, a1, u(0), u(0))
    l0, l1 = _tf2x32(b0, b1, u(0), u(0))
    higher = h0 ^ h1
    lower = l0 ^ l1
    mult = ((2 ** 16 % span) ** 2) % span
    off = ((higher % u(span)) * u(mult) + (lower % u(span))) % u(span)
    return off.astype(jnp.int32)


def _window(k0, k1, h, w):
    u = jnp.uint32
    cut_x = int(w * _RATIO + 0.5)
    cut_y = int(h * _RATIO + 0.5)
    ka0, ka1 = _tf2x32(k0, k1, u(0), u(0))
    kb0, kb1 = _tf2x32(k0, k1, u(0), u(1))
    ox = _randint_mod(ka0, ka1, w + (1 - cut_x % 2))
    oy = _randint_mod(kb0, kb1, h + (1 - cut_y % 2))
    x0 = jnp.maximum(ox - cut_x // 2, 0)
    x1 = jnp.minimum(ox - cut_x // 2 + cut_x - 1, w - 1)
    y0 = jnp.maximum(oy - cut_y // 2, 0)
    y1 = jnp.minimum(oy - cut_y // 2 + cut_y - 1, h - 1)
    return y0, y1, x0, x1


def kernel(x, key):
    h, w, c = x.shape
    n = h * w * c
    key_raw = jax.random.key_data(key).astype(jnp.uint32)
    xf = x.reshape(n)

    nw = 32
    per_w = n // nw
    ch = 49152
    nchunk = per_w // ch
    zb = 51 * c
    rows_per_w = h // nw

    mesh = plsc.VectorSubcoreMesh(core_axis_name="c", subcore_axis_name="s")

    @functools.partial(
        pl.kernel,
        mesh=mesh,
        out_type=jax.ShapeDtypeStruct((n,), x.dtype),
        scratch_types=[
            pltpu.VMEM((ch,), x.dtype),
            pltpu.VMEM((ch,), x.dtype),
            pltpu.VMEM((zb,), x.dtype),
            pltpu.VMEM((16,), jnp.uint32),
            pltpu.SemaphoreType.DMA,
            pltpu.SemaphoreType.DMA,
            pltpu.SemaphoreType.DMA,
            pltpu.SemaphoreType.DMA,
        ],
    )
    def sck(key_hbm, x_hbm, o_hbm, buf0, buf1, zbuf, keyv, li0, li1, so0, so1):
        wid = lax.axis_index("s") * 2 + lax.axis_index("c")
        base = wid * per_w
        pltpu.sync_copy(key_hbm, keyv.at[pl.ds(0, 2)])
        kv = keyv[pl.ds(0, 16)]
        k0 = kv[0]
        k1 = kv[1]
        y0, y1, x0, x1 = _window(k0, k1, h, w)

        def zinit(i, carry):
            zbuf[pl.ds(i * 16, 16)] = jnp.zeros((16,), x.dtype)
            return carry
        lax.fori_loop(0, zb // 16, zinit, 0)

        bufs = (buf0, buf1)
        lsems = (li0, li1)
        ssems = (so0, so1)
        loads = [None] * nchunk
        stores = [None] * nchunk
        for g in range(nchunk + 1):
            if g < nchunk:
                if g >= 2:
                    stores[g - 2].wait()
                loads[g] = pltpu.async_copy(
                    x_hbm.at[pl.ds(base + g * ch, ch)],
                    bufs[g % 2], lsems[g % 2])
            if g >= 1:
                loads[g - 1].wait()
                stores[g - 1] = pltpu.async_copy(
                    bufs[(g - 1) % 2],
                    o_hbm.at[pl.ds(base + (g - 1) * ch, ch)],
                    ssems[(g - 1) % 2])
        stores[nchunk - 2].wait()
        stores[nchunk - 1].wait()

        my_lo = jnp.maximum(y0, wid * rows_per_w)
        my_hi = jnp.minimum(y1, wid * rows_per_w + rows_per_w - 1)
        sx1 = x0 * c
        sx2 = (x1 + 1) * c - zb

        def zrow(r, carry):
            rowbase = r * (w * c)
            pltpu.sync_copy(zbuf, o_hbm.at[pl.ds(rowbase + sx1, zb)])
            pltpu.sync_copy(zbuf, o_hbm.at[pl.ds(rowbase + sx2, zb)])
            return carry
        lax.fori_loop(my_lo, my_hi + 1, zrow, 0)

    out = sck(key_raw, xf)
    return out.reshape(h, w, c)


# speedup vs baseline: 1.0215x; 1.0215x over previous
import functools

import jax
import jax.numpy as jnp
from jax import lax
from jax.experimental import pallas as pl
from jax.experimental.pallas import tpu as pltpu
from jax.experimental.pallas import tpu_sc as plsc

_RATIO = 0.2


def _tf2x32(k0, k1, c0, c1):
    u = jnp.uint32
    ks2 = k0 ^ k1 ^ u(0x1BD11BDA)
    ks = (k0, k1, ks2)
    x0 = c0 + k0
    x1 = c1 + k1
    rots = ((13, 15, 26, 6), (17, 29, 16, 24))
    for i in range(5):
        for r in rots[i % 2]:
            x0 = x0 + x1
            x1 = ((x1 << u(r)) | (x1 >> u(32 - r))) ^ x0
        x0 = x0 + ks[(i + 1) % 3]
        x1 = x1 + ks[(i + 2) % 3] + u(i + 1)
    return x0, x1


def _randint_mod(k0, k1, span):
    u = jnp.uint32
    a0, a1 = _tf2x32(k0, k1, u(0), u(0))
    b0, b1 = _tf2x32(k0, k1, u(0), u(1))
    h0, h1 = _tf2x32(a0, a1, u(0), u(0))
    l0, l1 = _tf2x32(b0, b1, u(0), u(0))
    higher = h0 ^ h1
    lower = l0 ^ l1
    mult = ((2 ** 16 % span) ** 2) % span
    off = ((higher % u(span)) * u(mult) + (lower % u(span))) % u(span)
    return off.astype(jnp.int32)


def _window(k0, k1, h, w):
    u = jnp.uint32
    cut_x = int(w * _RATIO + 0.5)
    cut_y = int(h * _RATIO + 0.5)
    ka0, ka1 = _tf2x32(k0, k1, u(0), u(0))
    kb0, kb1 = _tf2x32(k0, k1, u(0), u(1))
    ox = _randint_mod(ka0, ka1, w + (1 - cut_x % 2))
    oy = _randint_mod(kb0, kb1, h + (1 - cut_y % 2))
    x0 = jnp.maximum(ox - cut_x // 2, 0)
    x1 = jnp.minimum(ox - cut_x // 2 + cut_x - 1, w - 1)
    y0 = jnp.maximum(oy - cut_y // 2, 0)
    y1 = jnp.minimum(oy - cut_y // 2 + cut_y - 1, h - 1)
    return y0, y1, x0, x1


def kernel(x, key):
    h, w, c = x.shape
    n = h * w * c
    key_raw = jax.random.key_data(key).astype(jnp.uint32)
    xf = x.reshape(n)

    nw = 32
    per_w = n // nw
    ch = 49152
    nchunk = per_w // ch
    zb = 51 * c
    rows_per_w = h // nw

    mesh = plsc.VectorSubcoreMesh(core_axis_name="c", subcore_axis_name="s")

    @functools.partial(
        pl.kernel,
        mesh=mesh,
        out_type=jax.ShapeDtypeStruct((n,), x.dtype),
        scratch_types=[
            pltpu.VMEM_SHARED((16 * 2 * ch,), x.dtype),
            pltpu.VMEM((zb,), x.dtype),
            pltpu.VMEM((16,), jnp.uint32),
            pltpu.SemaphoreType.DMA,
            pltpu.SemaphoreType.DMA,
            pltpu.SemaphoreType.DMA,
            pltpu.SemaphoreType.DMA,
        ],
    )
    def sck(key_hbm, x_hbm, o_hbm, shared, zbuf, keyv, li0, li1, so0, so1):
        sid = lax.axis_index("s")
        wid = sid * 2 + lax.axis_index("c")
        base = wid * per_w
        buf0 = shared.at[pl.ds((sid * 2 + 0) * ch, ch)]
        buf1 = shared.at[pl.ds((sid * 2 + 1) * ch, ch)]
        pltpu.sync_copy(key_hbm, keyv.at[pl.ds(0, 2)])
        kv = keyv[pl.ds(0, 16)]
        k0 = kv[0]
        k1 = kv[1]
        y0, y1, x0, x1 = _window(k0, k1, h, w)

        def zinit(i, carry):
            zbuf[pl.ds(i * 16, 16)] = jnp.zeros((16,), x.dtype)
            return carry
        lax.fori_loop(0, zb // 16, zinit, 0)

        bufs = (buf0, buf1)
        lsems = (li0, li1)
        ssems = (so0, so1)
        loads = [None] * nchunk
        stores = [None] * nchunk
        for g in range(nchunk + 1):
            if g < nchunk:
                if g >= 2:
                    stores[g - 2].wait()
                loads[g] = pltpu.async_copy(
                    x_hbm.at[pl.ds(base + g * ch, ch)],
                    bufs[g % 2], lsems[g % 2])
            if g >= 1:
                loads[g - 1].wait()
                stores[g - 1] = pltpu.async_copy(
                    bufs[(g - 1) % 2],
                    o_hbm.at[pl.ds(base + (g - 1) * ch, ch)],
                    ssems[(g - 1) % 2])
        stores[nchunk - 2].wait()
        stores[nchunk - 1].wait()

        my_lo = jnp.maximum(y0, wid * rows_per_w)
        my_hi = jnp.minimum(y1, wid * rows_per_w + rows_per_w - 1)
        sx1 = x0 * c
        sx2 = (x1 + 1) * c - zb

        def zrow(r, carry):
            rowbase = r * (w * c)
            pltpu.sync_copy(zbuf, o_hbm.at[pl.ds(rowbase + sx1, zb)])
            pltpu.sync_copy(zbuf, o_hbm.at[pl.ds(rowbase + sx2, zb)])
            return carry
        lax.fori_loop(my_lo, my_hi + 1, zrow, 0)

    out = sck(key_raw, xf)
    return out.reshape(h, w, c)


# speedup vs baseline: 4.4678x; 4.3735x over previous
import jax
import jax.numpy as jnp
from jax.experimental import pallas as pl
from jax.experimental.pallas import tpu as pltpu

_RATIO = 0.2


def _tf2x32(k0, k1, c0, c1):
    u = jnp.uint32
    ks2 = k0 ^ k1 ^ u(0x1BD11BDA)
    ks = (k0, k1, ks2)
    x0 = c0 + k0
    x1 = c1 + k1
    rots = ((13, 15, 26, 6), (17, 29, 16, 24))
    for i in range(5):
        for r in rots[i % 2]:
            x0 = x0 + x1
            x1 = ((x1 << u(r)) | (x1 >> u(32 - r))) ^ x0
        x0 = x0 + ks[(i + 1) % 3]
        x1 = x1 + ks[(i + 2) % 3] + u(i + 1)
    return x0, x1


def _randint_mod(k0, k1, span):
    u = jnp.uint32
    a0, a1 = _tf2x32(k0, k1, u(0), u(0))
    b0, b1 = _tf2x32(k0, k1, u(0), u(1))
    h0, h1 = _tf2x32(a0, a1, u(0), u(0))
    l0, l1 = _tf2x32(b0, b1, u(0), u(0))
    higher = h0 ^ h1
    lower = l0 ^ l1
    mult = ((2 ** 16 % span) ** 2) % span
    off = ((higher % u(span)) * u(mult) + (lower % u(span))) % u(span)
    return off.astype(jnp.int32)


def _window(key_ref, h, w):
    u = jnp.uint32
    cut_x = int(w * _RATIO + 0.5)
    cut_y = int(h * _RATIO + 0.5)
    k0, k1 = key_ref[0], key_ref[1]
    ka0, ka1 = _tf2x32(k0, k1, u(0), u(0))
    kb0, kb1 = _tf2x32(k0, k1, u(0), u(1))
    ox = _randint_mod(ka0, ka1, w + (1 - cut_x % 2))
    oy = _randint_mod(kb0, kb1, h + (1 - cut_y % 2))
    x0 = jnp.maximum(ox - cut_x // 2, 0)
    x1 = jnp.minimum(ox - cut_x // 2 + cut_x - 1, w - 1)
    y0 = jnp.maximum(oy - cut_y // 2, 0)
    y1 = jnp.minimum(oy - cut_y // 2 + cut_y - 1, h - 1)
    return y0, y1, x0, x1


def _body(key_ref, x_ref, o_ref):
    by, w, c = x_ref.shape
    h = pl.num_programs(0) * by
    i = pl.program_id(0)
    r0 = i * by
    y0, y1, x0, x1 = _window(key_ref, h, w)
    intersects = (r0 <= y1) & (r0 + by - 1 >= y0)

    @pl.when(intersects)
    def _masked():
        rows = r0 + jax.lax.broadcasted_iota(jnp.int32, (by, w, c), 0)
        cols = jax.lax.broadcasted_iota(jnp.int32, (by, w, c), 1)
        inside = (rows >= y0) & (rows <= y1) & (cols >= x0) & (cols <= x1)
        o_ref[...] = jnp.where(inside, jnp.zeros_like(o_ref), x_ref[...])

    @pl.when(jnp.logical_not(intersects))
    def _copy():
        o_ref[...] = x_ref[...]


def kernel(x, key):
    h, w, c = x.shape
    key_raw = jax.random.key_data(key).astype(jnp.uint32)
    by = 16
    return pl.pallas_call(
        _body,
        grid=(h // by,),
        in_specs=[
            pl.BlockSpec(memory_space=pltpu.SMEM),
            pl.BlockSpec((by, w, c), lambda i: (i, 0, 0)),
        ],
        out_specs=pl.BlockSpec((by, w, c), lambda i: (i, 0, 0)),
        out_shape=jax.ShapeDtypeStruct((h, w, c), x.dtype),
    )(key_raw, x)
